# Initial kernel scaffold; baseline (speedup 1.0000x reference)
import jax
import jax.numpy as jnp
from jax.experimental import pallas as pl


def kernel(x_fake, densities, bin_min, bin_max):
    raise NotImplementedError("write your pallas kernel here")


# speedup vs baseline: 18.6976x; 18.6976x over previous
import functools

import jax
import jax.numpy as jnp
from jax import lax
from jax.experimental import pallas as pl
from jax.experimental.pallas import tpu as pltpu
from jax.experimental.pallas import tpu_sc as plsc

N_BINS = 64
NC, NS, L = 2, 16, 16
NW = NC * NS


def _sc_hist(x2d, ab, B, C):
    rows_per_w = B // NW
    n_groups = (C + L - 1) // L
    table_len = C * N_BINS

    mesh = plsc.VectorSubcoreMesh(
        core_axis_name="c", subcore_axis_name="s", num_cores=NC,
        num_subcores=NS)

    @functools.partial(
        pl.kernel,
        out_type=jax.ShapeDtypeStruct((NW, table_len), jnp.float32),
        mesh=mesh,
        scratch_types=[
            pltpu.VMEM((rows_per_w // 2, C), jnp.float32),
            pltpu.VMEM((2, C), jnp.float32),
            pltpu.VMEM((table_len,), jnp.float32),
        ],
        compiler_params=pltpu.CompilerParams(needs_layout_passes=False),
    )
    def hist_kernel(x_hbm, ab_hbm, out_hbm, x_buf, ab_buf, table):
        wid = lax.axis_index("s") * NC + lax.axis_index("c")
        base_row = wid * rows_per_w

        pltpu.sync_copy(ab_hbm, ab_buf)

        zeros = jnp.zeros((L,), jnp.float32)

        def zero_body(i, _):
            table[pl.ds(pl.multiple_of(i * L, L), L)] = zeros
            return _

        lax.fori_loop(0, table_len // L, zero_body, None)

        lane = lax.iota(jnp.int32, L)
        ones = jnp.ones((L,), jnp.float32)
        chunk = rows_per_w // 2

        for half in range(2):
            pltpu.sync_copy(
                x_hbm.at[pl.ds(base_row + half * chunk, chunk), :], x_buf)

            for g in range(n_groups):
                off = C - L if g == n_groups - 1 else g * L
                full = (g + 1) * L <= C
                av = ab_buf[0, pl.ds(off, L)]
                bv = ab_buf[1, pl.ds(off, L)]
                sv = float(N_BINS) / (bv - av)
                base = (lane + off) * N_BINS
                gmask = None if full else lane >= (g * L - off)

                def row_body(r, _):
                    xv = x_buf[r, pl.ds(off, L)]
                    u = (xv - av) * sv
                    cl = jnp.minimum(jnp.maximum(u, 0.0), float(N_BINS - 1))
                    idx = cl.astype(jnp.int32) + base
                    valid = (xv >= av) & (xv <= bv)
                    if gmask is not None:
                        valid = valid & gmask
                    plsc.addupdate_scatter(table, [idx], ones, mask=valid)
                    return _

                lax.fori_loop(0, chunk, row_body, None)

        pltpu.sync_copy(table, out_hbm.at[wid])

    return hist_kernel(x2d, ab)


def _tc_loss(tables3, dens3, B, n_entries):
    def loss_body(tabs_ref, dens_ref, out_ref):
        counts = jnp.sum(tabs_ref[...], axis=0)
        diff = jnp.abs(counts * (1.0 / B) - dens_ref[...])
        out_ref[0, 0] = jnp.sum(diff) * (1.0 / n_entries)

    out = pl.pallas_call(
        loss_body,
        out_shape=jax.ShapeDtypeStruct((1, 1), jnp.float32),
        out_specs=pl.BlockSpec(memory_space=pltpu.SMEM),
    )(tables3, dens3)
    return out[0, 0]


def kernel(x_fake, densities, bin_min, bin_max):
    B, T, D = x_fake.shape
    C = T * D
    x2d = x_fake.reshape(B, C)
    ab = jnp.stack([bin_min, bin_max])
    tables = _sc_hist(x2d, ab, B, C)
    tables3 = tables.reshape(NW, C * N_BINS // 128, 128)
    dens3 = densities.reshape(C * N_BINS // 128, 128)
    return _tc_loss(tables3, dens3, B, C * N_BINS)
